# Initial kernel scaffold; baseline (speedup 1.0000x reference)
import jax
import jax.numpy as jnp
from jax.experimental import pallas as pl


def kernel(x, edge_index, W1, b1, W2, b2):
    raise NotImplementedError("write your pallas kernel here")


# speedup vs baseline: 12.8809x; 12.8809x over previous
import functools

import jax
import jax.numpy as jnp
from jax import lax
from jax.experimental import pallas as pl
from jax.experimental.pallas import tpu as pltpu
from jax.experimental.pallas import tpu_sc as plsc

N_NODE = 10000
D_FEAT = 128
D_OUT_PAD = 16

NC, NS = 2, 16
NW = NC * NS
NP = 10240
CH = 128
JPT = 79
E_PAD = NW * JPT * CH
ROWS_PT = NP // NS
RB = 512


def _mesh():
  return plsc.VectorSubcoreMesh(core_axis_name="c", subcore_axis_name="s",
                                num_cores=NC, num_subcores=NS)


_SC_PARAMS = pltpu.CompilerParams(use_tc_tiling_on_sc=False)


def _worker_id():
  return lax.axis_index("s") * NC + lax.axis_index("c")


def _deg_body(dst_hbm, ones_hbm, zeros_hbm, out_hbm, idx_d, ones_v, acc):
  c = lax.axis_index("c")
  s = lax.axis_index("s")
  wid = _worker_id()
  row0 = s * ROWS_PT
  pltpu.sync_copy(zeros_hbm.at[pl.ds(row0, ROWS_PT)], acc.at[pl.ds(row0, ROWS_PT)])
  pltpu.sync_copy(ones_hbm, ones_v)
  plsc.subcore_barrier()

  def chunk(j, _):
    base = (wid * JPT + j) * CH
    pltpu.sync_copy(dst_hbm.at[pl.ds(base, CH)], idx_d)
    pltpu.sync_copy(ones_v, acc.at[idx_d], add=True)
    return 0

  lax.fori_loop(0, JPT, chunk, 0)
  plsc.subcore_barrier()
  pltpu.sync_copy(acc.at[pl.ds(row0, ROWS_PT)], out_hbm.at[c, pl.ds(row0, ROWS_PT)])


def _deg_kernel(dst):
  ones = jnp.ones((CH, 8), jnp.float32)
  zeros = jnp.zeros((NP, 8), jnp.float32)
  k = functools.partial(
      pl.kernel,
      out_type=jax.ShapeDtypeStruct((NC, NP, 8), jnp.float32),
      mesh=_mesh(),
      compiler_params=_SC_PARAMS,
      scratch_types=[
          pltpu.VMEM((CH,), jnp.int32),
          pltpu.VMEM((CH, 8), jnp.float32),
          pltpu.VMEM_SHARED((NP, 8), jnp.float32),
      ],
  )(_deg_body)
  return k(dst, ones, zeros)


def _make_scatter_body(width):
  def body(table_hbm, src_hbm, dst_hbm, zeros_hbm, out_hbm,
           idx_s, idx_d, rows, acc, sem):
    c = lax.axis_index("c")
    s = lax.axis_index("s")
    wid = _worker_id()
    row0 = s * ROWS_PT
    pltpu.sync_copy(zeros_hbm.at[pl.ds(row0, ROWS_PT)],
                    acc.at[pl.ds(row0, ROWS_PT)])
    plsc.subcore_barrier()

    def chunk(j, _):
      base = (wid * JPT + j) * CH
      pltpu.sync_copy(src_hbm.at[pl.ds(base, CH)], idx_s)
      pltpu.sync_copy(dst_hbm.at[pl.ds(base, CH)], idx_d)
      pltpu.async_copy(table_hbm.at[idx_s], rows, sem).wait()
      pltpu.sync_copy(rows, acc.at[idx_d], add=True)
      return 0

    lax.fori_loop(0, JPT, chunk, 0)
    plsc.subcore_barrier()
    pltpu.sync_copy(acc.at[pl.ds(row0, ROWS_PT)],
                    out_hbm.at[c, pl.ds(row0, ROWS_PT)])

  return body


def _scatter_kernel(table, src, dst, width):
  zeros = jnp.zeros((NP, width), jnp.float32)
  k = functools.partial(
      pl.kernel,
      out_type=jax.ShapeDtypeStruct((NC, NP, width), jnp.float32),
      mesh=_mesh(),
      compiler_params=_SC_PARAMS,
      scratch_types=[
          pltpu.VMEM((CH,), jnp.int32),
          pltpu.VMEM((CH,), jnp.int32),
          pltpu.VMEM((CH, width), jnp.float32),
          pltpu.VMEM_SHARED((NP, width), jnp.float32),
          pltpu.SemaphoreType.DMA,
      ],
  )(_make_scatter_body(width))
  return k(table, src, dst, zeros)


def _tc1_body(x_ref, w_ref, pa_ref, pb_ref, g_ref, d_ref):
  d = lax.rsqrt(pa_ref[...] + pb_ref[...] + 1.0)
  h = jnp.dot(x_ref[...], w_ref[...], preferred_element_type=jnp.float32)
  g_ref[...] = h * d[:, None]
  d_ref[...] = d


def _tc1(xp, W1, pa, pb):
  return pl.pallas_call(
      _tc1_body,
      grid=(NP // RB,),
      in_specs=[
          pl.BlockSpec((RB, D_FEAT), lambda i: (i, 0)),
          pl.BlockSpec((D_FEAT, D_FEAT), lambda i: (0, 0)),
          pl.BlockSpec((RB,), lambda i: (i,)),
          pl.BlockSpec((RB,), lambda i: (i,)),
      ],
      out_specs=[
          pl.BlockSpec((RB, D_FEAT), lambda i: (i, 0)),
          pl.BlockSpec((RB,), lambda i: (i,)),
      ],
      out_shape=[
          jax.ShapeDtypeStruct((NP, D_FEAT), jnp.float32),
          jax.ShapeDtypeStruct((NP,), jnp.float32),
      ],
  )(xp, W1, pa, pb)


def _tc2_body(g_ref, sa_ref, sb_ref, d_ref, b1_ref, w2_ref, g2_ref):
  d = d_ref[...]
  z = d[:, None] * (sa_ref[...] + sb_ref[...] + g_ref[...]) + b1_ref[...][None, :]
  z = jnp.maximum(z, 0.0)
  h2 = jnp.dot(z, w2_ref[...], preferred_element_type=jnp.float32)
  g2_ref[...] = h2 * d[:, None]


def _tc2(g, sa, sb, d, b1, W2p):
  return pl.pallas_call(
      _tc2_body,
      grid=(NP // RB,),
      in_specs=[
          pl.BlockSpec((RB, D_FEAT), lambda i: (i, 0)),
          pl.BlockSpec((RB, D_FEAT), lambda i: (i, 0)),
          pl.BlockSpec((RB, D_FEAT), lambda i: (i, 0)),
          pl.BlockSpec((RB,), lambda i: (i,)),
          pl.BlockSpec((D_FEAT,), lambda i: (0,)),
          pl.BlockSpec((D_FEAT, D_OUT_PAD), lambda i: (0, 0)),
      ],
      out_specs=pl.BlockSpec((RB, D_OUT_PAD), lambda i: (i, 0)),
      out_shape=jax.ShapeDtypeStruct((NP, D_OUT_PAD), jnp.float32),
  )(g, sa, sb, d, b1, W2p)


def _tc3_body(g2_ref, sa_ref, sb_ref, d_ref, b2_ref, y_ref):
  d = d_ref[...]
  y = d[:, None] * (sa_ref[...] + sb_ref[...] + g2_ref[...]) + b2_ref[...][None, :]
  y_ref[...] = jnp.tanh(jnp.maximum(y, 0.0))


def _tc3(g2, sa, sb, d, b2p):
  return pl.pallas_call(
      _tc3_body,
      grid=(NP // RB,),
      in_specs=[
          pl.BlockSpec((RB, D_OUT_PAD), lambda i: (i, 0)),
          pl.BlockSpec((RB, D_OUT_PAD), lambda i: (i, 0)),
          pl.BlockSpec((RB, D_OUT_PAD), lambda i: (i, 0)),
          pl.BlockSpec((RB,), lambda i: (i,)),
          pl.BlockSpec((D_OUT_PAD,), lambda i: (0,)),
      ],
      out_specs=pl.BlockSpec((RB, D_OUT_PAD), lambda i: (i, 0)),
      out_shape=jax.ShapeDtypeStruct((NP, D_OUT_PAD), jnp.float32),
  )(g2, sa, sb, d, b2p)


def kernel(x, edge_index, W1, b1, W2, b2):
  ei = edge_index.astype(jnp.int32)
  pad = jnp.full((E_PAD - ei.shape[1],), N_NODE, jnp.int32)
  src = jnp.concatenate([ei[0], pad])
  dst = jnp.concatenate([ei[1], pad])
  xp = jnp.pad(x, ((0, NP - N_NODE), (0, 0)))
  W2p = jnp.pad(W2, ((0, 0), (0, D_OUT_PAD - W2.shape[1])))
  b2p = jnp.pad(b2, (0, D_OUT_PAD - b2.shape[0]))

  degp = _deg_kernel(dst)
  g, d = _tc1(xp, W1, degp[0, :, 0], degp[1, :, 0])
  sp = _scatter_kernel(g, src, dst, D_FEAT)
  g2 = _tc2(g, sp[0], sp[1], d, b1, W2p)
  s2p = _scatter_kernel(g2, src, dst, D_OUT_PAD)
  y = _tc3(g2, s2p[0], s2p[1], d, b2p)
  return y[:N_NODE, :3]
